# baseline (speedup 1.0000x reference)
import jax
import jax.numpy as jnp
from jax import lax
from jax.experimental import pallas as pl
from jax.experimental.pallas import tpu as pltpu
from jax.experimental.pallas import tpu_sc as plsc

N_NODES = 10000
N_EDGES = 320000
EMB = 128
CAT = 16
K_HOPS = 3
ALPHA = 1.0 / (K_HOPS + 1)

NC = 2
NS = 16
NW = NC * NS
EPW = N_EDGES // NW
CHUNK = 80
NCHUNK = EPW // CHUNK
DEGW = 16

BR = 400
GRID = N_NODES // BR


def _mesh():
    return plsc.VectorSubcoreMesh(
        core_axis_name="c", subcore_axis_name="s",
        num_cores=NC, num_subcores=NS)


def _sc_prep(src3, cat3, dst3, zerosn1):
    def body(src_h, cat_h, dst_h, zeros_h, gidx_out, degp_out,
             src_v, cat_v, dst_v, gidx_v, deg_v):
        cid = lax.axis_index("c")
        sid = lax.axis_index("s")
        wid = sid * NC + cid
        pltpu.sync_copy(src_h.at[wid], src_v)
        pltpu.sync_copy(cat_h.at[wid], cat_v)
        pltpu.sync_copy(dst_h.at[wid], dst_v)
        pltpu.sync_copy(zeros_h, deg_v)
        ones16 = jnp.ones((16,), jnp.float32)

        def chunk(i, carry):
            for j in range(CHUNK // 16):
                sl = pl.ds(j * 16, 16)
                gidx_v[i, sl] = src_v[i, sl] * CAT + cat_v[i, sl]
                plsc.addupdate_scatter(deg_v, [dst_v[i, sl]], ones16)
            return carry

        lax.fori_loop(0, NCHUNK, chunk, 0)
        pltpu.sync_copy(gidx_v, gidx_out.at[wid])
        pltpu.sync_copy(deg_v, degp_out.at[wid])

    kfn = pl.kernel(
        body,
        out_type=[
            jax.ShapeDtypeStruct((NW, NCHUNK, CHUNK), jnp.int32),
            jax.ShapeDtypeStruct((NW, N_NODES), jnp.float32),
        ],
        mesh=_mesh(),
        compiler_params=pltpu.CompilerParams(needs_layout_passes=False),
        scratch_types=[
            pltpu.VMEM((NCHUNK, CHUNK), jnp.int32),
            pltpu.VMEM((NCHUNK, CHUNK), jnp.int32),
            pltpu.VMEM((NCHUNK, CHUNK), jnp.int32),
            pltpu.VMEM((NCHUNK, CHUNK), jnp.int32),
            pltpu.VMEM((N_NODES,), jnp.float32),
        ],
    )
    return kfn(src3, cat3, dst3, zerosn1)


def _sc_hop(g2, gidx1, dst1, zeros_n):
    def body(g_h, gidx1_h, dst1_h, zeros_h, p_out,
             gidxc_v, dstc_v, rows_v, acc_sh, sem1, sem2, sem3):
        cid = lax.axis_index("c")
        sid = lax.axis_index("s")
        wid = sid * NC + cid

        @pl.when(sid == 0)
        def _():
            pltpu.sync_copy(zeros_h, acc_sh)

        plsc.subcore_barrier()

        def chunk(i, carry):
            off = wid * EPW + i * CHUNK
            pltpu.async_copy(gidx1_h.at[pl.ds(off, CHUNK)], gidxc_v, sem1).wait()
            pltpu.async_copy(dst1_h.at[pl.ds(off, CHUNK)], dstc_v, sem1).wait()
            pltpu.async_copy(g_h.at[gidxc_v], rows_v, sem2).wait()
            pltpu.async_copy(rows_v, acc_sh.at[dstc_v], sem3, add=True).wait()
            return carry

        for ph in range(NS):
            @pl.when(sid == ph)
            def _():
                lax.fori_loop(0, NCHUNK, chunk, 0)
            plsc.subcore_barrier()

        @pl.when(sid == 0)
        def _():
            pltpu.sync_copy(acc_sh, p_out.at[cid])

    kfn = pl.kernel(
        body,
        out_type=[jax.ShapeDtypeStruct((NC, N_NODES, EMB), jnp.float32)],
        mesh=_mesh(),
        compiler_params=pltpu.CompilerParams(needs_layout_passes=False),
        scratch_types=[
            pltpu.VMEM((CHUNK,), jnp.int32),
            pltpu.VMEM((CHUNK,), jnp.int32),
            pltpu.VMEM((CHUNK, EMB), jnp.float32),
            pltpu.VMEM_SHARED((N_NODES, EMB), jnp.float32),
            pltpu.SemaphoreType.DMA,
            pltpu.SemaphoreType.DMA,
            pltpu.SemaphoreType.DMA,
        ],
    )
    return kfn(g2, gidx1, dst1, zeros_n)[0]


def _tc_prep(degp, node_emb, cat_table):
    def body(degp_ref, ne_ref, cat_ref, dr_ref, g_ref, hacc_ref):
        deg = jnp.sum(degp_ref[...], axis=0)
        dr = lax.rsqrt(jnp.maximum(deg, 1.0))
        dr_ref[...] = dr
        ne = ne_ref[...]
        m = ne * dr
        g_ref[...] = m[:, None, :] * cat_ref[...][None, :, :]
        hacc_ref[...] = ALPHA * ne

    return pl.pallas_call(
        body,
        grid=(GRID,),
        in_specs=[
            pl.BlockSpec((NW, BR, 1), lambda i: (0, i, 0)),
            pl.BlockSpec((BR, EMB), lambda i: (i, 0)),
            pl.BlockSpec((CAT, EMB), lambda i: (0, 0)),
        ],
        out_specs=[
            pl.BlockSpec((BR, 1), lambda i: (i, 0)),
            pl.BlockSpec((BR, CAT, EMB), lambda i: (i, 0, 0)),
            pl.BlockSpec((BR, EMB), lambda i: (i, 0)),
        ],
        out_shape=[
            jax.ShapeDtypeStruct((N_NODES, 1), jnp.float32),
            jax.ShapeDtypeStruct((N_NODES, CAT, EMB), jnp.float32),
            jax.ShapeDtypeStruct((N_NODES, EMB), jnp.float32),
        ],
    )(degp.reshape(NW, N_NODES, 1), node_emb, cat_table)


def _tc_update(p, dr, hacc, cat_table):
    def body(p_ref, dr_ref, hacc_ref, cat_ref, hacc_out, g_ref):
        s = p_ref[0] + p_ref[1]
        dr = dr_ref[...]
        h = dr * s
        hacc_out[...] = hacc_ref[...] + ALPHA * h
        m = dr * h
        g_ref[...] = m[:, None, :] * cat_ref[...][None, :, :]

    return pl.pallas_call(
        body,
        grid=(GRID,),
        in_specs=[
            pl.BlockSpec((2, BR, EMB), lambda i: (0, i, 0)),
            pl.BlockSpec((BR, 1), lambda i: (i, 0)),
            pl.BlockSpec((BR, EMB), lambda i: (i, 0)),
            pl.BlockSpec((CAT, EMB), lambda i: (0, 0)),
        ],
        out_specs=[
            pl.BlockSpec((BR, EMB), lambda i: (i, 0)),
            pl.BlockSpec((BR, CAT, EMB), lambda i: (i, 0, 0)),
        ],
        out_shape=[
            jax.ShapeDtypeStruct((N_NODES, EMB), jnp.float32),
            jax.ShapeDtypeStruct((N_NODES, CAT, EMB), jnp.float32),
        ],
    )(p, dr, hacc, cat_table)


def _tc_final(p, dr, hacc, node_emb, cx, wih_t, whh_t, bih, bhh):
    def body(p_ref, dr_ref, hacc_ref, ne_ref, cx_ref, wih_ref, whh_ref,
             bih_ref, bhh_ref, hn_ref):
        s = p_ref[0] + p_ref[1]
        hacc = hacc_ref[...] + ALPHA * (dr_ref[...] * s)
        ne = ne_ref[...]
        gates = (
            jnp.dot(hacc, wih_ref[...], preferred_element_type=jnp.float32,
                    precision=lax.Precision.HIGHEST)
            + jnp.dot(ne, whh_ref[...], preferred_element_type=jnp.float32,
                      precision=lax.Precision.HIGHEST)
            + bih_ref[...] + bhh_ref[...])
        i_g = jax.nn.sigmoid(gates[:, 0:EMB])
        f_g = jax.nn.sigmoid(gates[:, EMB:2 * EMB])
        g_g = jnp.tanh(gates[:, 2 * EMB:3 * EMB])
        o_g = jax.nn.sigmoid(gates[:, 3 * EMB:4 * EMB])
        cn = f_g * cx_ref[...] + i_g * g_g
        hn_ref[...] = o_g * jnp.tanh(cn)

    return pl.pallas_call(
        body,
        grid=(GRID,),
        in_specs=[
            pl.BlockSpec((2, BR, EMB), lambda i: (0, i, 0)),
            pl.BlockSpec((BR, 1), lambda i: (i, 0)),
            pl.BlockSpec((BR, EMB), lambda i: (i, 0)),
            pl.BlockSpec((BR, EMB), lambda i: (i, 0)),
            pl.BlockSpec((BR, EMB), lambda i: (i, 0)),
            pl.BlockSpec((EMB, 4 * EMB), lambda i: (0, 0)),
            pl.BlockSpec((EMB, 4 * EMB), lambda i: (0, 0)),
            pl.BlockSpec((1, 4 * EMB), lambda i: (0, 0)),
            pl.BlockSpec((1, 4 * EMB), lambda i: (0, 0)),
        ],
        out_specs=pl.BlockSpec((BR, EMB), lambda i: (i, 0)),
        out_shape=jax.ShapeDtypeStruct((N_NODES, EMB), jnp.float32),
    )(p, dr, hacc, node_emb, cx, wih_t, whh_t, bih, bhh)


def kernel(node_emb, cx, cat_table, W_ih, W_hh, b_ih, b_hh, edge_index, cat_idx):
    src = edge_index[0].astype(jnp.int32).reshape(NW, NCHUNK, CHUNK)
    cat = cat_idx.astype(jnp.int32).reshape(NW, NCHUNK, CHUNK)
    dst = edge_index[1].astype(jnp.int32).reshape(NW, NCHUNK, CHUNK)
    zerosn1 = jnp.zeros((N_NODES,), jnp.float32)
    zeros_n = jnp.zeros((N_NODES, EMB), jnp.float32)
    wih_t = W_ih.T
    whh_t = W_hh.T
    bih = b_ih.reshape(1, 4 * EMB)
    bhh = b_hh.reshape(1, 4 * EMB)

    gidx3, degp = _sc_prep(src, cat, dst, zerosn1)
    dr, g3, hacc = _tc_prep(degp, node_emb, cat_table)
    hn = None
    for hop in range(K_HOPS):
        p = _sc_hop(g3.reshape(N_NODES * CAT, EMB), gidx3.reshape(-1),
                    dst.reshape(-1), zeros_n)
        if hop < K_HOPS - 1:
            hacc, g3 = _tc_update(p, dr, hacc, cat_table)
        else:
            hn = _tc_final(p, dr, hacc, node_emb, cx, wih_t, whh_t, bih, bhh)
    return hn
